# baseline (speedup 1.0000x reference)
import functools

import jax
import jax.numpy as jnp
from jax import lax
from jax.experimental import pallas as pl
from jax.experimental.pallas import tpu as pltpu
from jax.experimental.pallas import tpu_sc as plsc

NC = 2
NS = 16
NW = NC * NS
CH = 128

G = 64


def _mesh():
    return plsc.VectorSubcoreMesh(
        core_axis_name="c", subcore_axis_name="s", num_cores=NC, num_subcores=NS
    )


def _make_deg(np_, nch, rpt):
    @functools.partial(
        pl.kernel,
        out_type=jax.ShapeDtypeStruct((NC, np_, 16), jnp.float32),
        mesh=_mesh(),
        scratch_types=[
            pltpu.VMEM((nch, CH), jnp.int32),
            pltpu.VMEM((CH, 16), jnp.float32),
            pltpu.VMEM_SHARED((np_, 16), jnp.float32),
        ],
    )
    def deg_kernel(dst_hbm, z_hbm, out_hbm, dst_v, ones_v, acc):
        cid = lax.axis_index("c")
        sid = lax.axis_index("s")
        wid = sid * NC + cid
        pltpu.sync_copy(z_hbm, acc.at[pl.ds(sid * rpt, rpt)])
        pltpu.sync_copy(dst_hbm.at[pl.ds(wid * nch, nch)], dst_v)

        def init_ones(i, _):
            ones_v[i, :] = jnp.ones((16,), jnp.float32)
            return 0

        lax.fori_loop(0, CH, init_ones, 0)
        plsc.subcore_barrier()

        def body(j, _):
            pltpu.sync_copy(ones_v, acc.at[dst_v.at[j]], add=True)
            return 0

        lax.fori_loop(0, nch, body, 0)
        plsc.subcore_barrier()
        pltpu.sync_copy(acc.at[pl.ds(sid * rpt, rpt)],
                        out_hbm.at[cid, pl.ds(sid * rpt, rpt)])

    return deg_kernel


def _make_prop(np_, nch0, nch1, rpt, d):
    nmax = max(nch0, nch1)

    @functools.partial(
        pl.kernel,
        out_type=jax.ShapeDtypeStruct((NC, np_, d), jnp.float32),
        mesh=_mesh(),
        scratch_types=[
            pltpu.VMEM((nmax, CH), jnp.int32),
            pltpu.VMEM((nmax, CH), jnp.int32),
            [pltpu.VMEM((CH, d), jnp.float32) for _ in range(2)],
            pltpu.VMEM_SHARED((np_, d), jnp.float32),
            pltpu.SemaphoreType.DMA,
        ],
    )
    def prop_kernel(src_hbm, dst_hbm, s_hbm, z_hbm, out_hbm,
                    src_v, dst_v, rows, acc, sem):
        cid = lax.axis_index("c")
        sid = lax.axis_index("s")
        base = jnp.where(cid == 0, sid * nch0, NS * nch0 + sid * nch1)
        npairs = jnp.where(cid == 0, nch0 // 2, nch1 // 2)
        pltpu.sync_copy(z_hbm, acc.at[pl.ds(sid * rpt, rpt)])
        pltpu.sync_copy(src_hbm.at[pl.ds(base, nmax)], src_v)
        pltpu.sync_copy(dst_hbm.at[pl.ds(base, nmax)], dst_v)
        plsc.subcore_barrier()

        def body(k, _):
            jj = 2 * k
            descs = [
                pltpu.async_copy(s_hbm.at[src_v.at[jj + b]], rows[b], sem)
                for b in range(2)
            ]
            for b in range(2):
                descs[b].wait()
            for b in range(2):
                pltpu.sync_copy(rows[b], acc.at[dst_v.at[jj + b]], add=True)
            return 0

        lax.fori_loop(0, npairs, body, 0)
        plsc.subcore_barrier()
        pltpu.sync_copy(acc.at[pl.ds(sid * rpt, rpt)],
                        out_hbm.at[cid, pl.ds(sid * rpt, rpt)])

    return prop_kernel


def _elu(z):
    return jnp.where(z > 0, z, jnp.exp(z) - 1.0)


def _full(shape):
    return pl.BlockSpec(shape, lambda *i: (0,) * len(shape))


def _rows(shape):
    return pl.BlockSpec(shape, lambda i: (i,) + (0,) * (len(shape) - 1))


def _tc0(x, w1, da, db, bn, wpad):
    np_, din = x.shape
    dout = w1.shape[1]

    def body(x_ref, w_ref, da_ref, db_ref, s_ref, dv_ref):
        deg = 1.0 + da_ref[:, :1] + db_ref[:, :1]
        dinv = lax.rsqrt(deg)
        s = dinv * jnp.dot(x_ref[...], w_ref[...],
                           preferred_element_type=jnp.float32)
        s_ref[...] = jnp.concatenate(
            [s, jnp.zeros((bn, wpad - dout), jnp.float32)], axis=1)
        dv_ref[...] = dinv

    return pl.pallas_call(
        body,
        grid=(np_ // bn,),
        in_specs=[_rows((bn, din)), _full(w1.shape), _rows((bn, 16)),
                  _rows((bn, 16))],
        out_specs=[_rows((bn, wpad)), _rows((bn, 1))],
        out_shape=[jax.ShapeDtypeStruct((np_, wpad), jnp.float32),
                   jax.ShapeDtypeStruct((np_, 1), jnp.float32)],
    )(x, w1, da, db)


def _tc_layer1(ea, eb, s1, dinv, b1, bn):
    np_, wpad = s1.shape
    d = b1.shape[1]

    def body(ea_ref, eb_ref, s_ref, dv_ref, b_ref, o_ref):
        dinv_ = dv_ref[...]
        agg = (ea_ref[:, :d] + eb_ref[:, :d] + s_ref[:, :d])
        z = dinv_ * agg + b_ref[...]
        s2 = dinv_ * _elu(z)
        o_ref[...] = jnp.concatenate(
            [s2, jnp.zeros((bn, wpad - d), jnp.float32)], axis=1)

    return pl.pallas_call(
        body,
        grid=(np_ // bn,),
        in_specs=[_rows((bn, wpad)), _rows((bn, wpad)), _rows((bn, wpad)),
                  _rows((bn, 1)), _full(b1.shape)],
        out_specs=_rows((bn, wpad)),
        out_shape=jax.ShapeDtypeStruct((np_, wpad), jnp.float32),
    )(ea, eb, s1, dinv, b1)


def _tc_layer(ea, eb, s, dinv, w, b, bn, scale_out, wpad_out):
    np_, wpad_in = s.shape
    din, dout = w.shape

    def body(ea_ref, eb_ref, s_ref, dv_ref, w_ref, b_ref, o_ref):
        dinv_ = dv_ref[...]
        agg = dinv_ * (ea_ref[:, :din] + eb_ref[:, :din] + s_ref[:, :din])
        z = jnp.dot(agg, w_ref[...], preferred_element_type=jnp.float32) + b_ref[...]
        h = _elu(z)
        h = dinv_ * h if scale_out else h
        if wpad_out > dout:
            h = jnp.concatenate(
                [h, jnp.zeros((bn, wpad_out - dout), jnp.float32)], axis=1)
        o_ref[...] = h

    return pl.pallas_call(
        body,
        grid=(np_ // bn,),
        in_specs=[_rows((bn, wpad_in)), _rows((bn, wpad_in)),
                  _rows((bn, wpad_in)), _rows((bn, 1)), _full(w.shape),
                  _full(b.shape)],
        out_specs=_rows((bn, wpad_out)),
        out_shape=jax.ShapeDtypeStruct((np_, wpad_out), jnp.float32),
    )(ea, eb, s, dinv, w, b)


def _tc_head(h5, batch2, lw1, lb1, lw2, lb2):
    np_, d = h5.shape
    ncls = lw2.shape[1]

    def body(h_ref, b_ref, w1_ref, b1_ref, w2_ref, b2_ref, o_ref):
        gids = lax.broadcasted_iota(jnp.int32, (G, np_), 0)
        onehot = (gids == b_ref[...]).astype(jnp.float32)
        sums = jnp.dot(onehot, h_ref[...], preferred_element_type=jnp.float32)
        cnt = jnp.sum(onehot, axis=1, keepdims=True)
        pooled = sums / jnp.maximum(cnt, 1.0)
        y = _elu(jnp.dot(pooled, w1_ref[...],
                         preferred_element_type=jnp.float32) + b1_ref[...])
        o_ref[...] = jnp.dot(y, w2_ref[...],
                             preferred_element_type=jnp.float32) + b2_ref[...]

    return pl.pallas_call(
        body,
        in_specs=[_full(h5.shape), _full(batch2.shape), _full(lw1.shape),
                  _full(lb1.shape), _full(lw2.shape), _full(lb2.shape)],
        out_specs=_full((G, ncls)),
        out_shape=jax.ShapeDtypeStruct((G, ncls), jnp.float32),
    )(h5, batch2, lw1, lb1, lw2, lb2)


def kernel(x, edge_index, batch, W1, b1, W2, b2, W3, b3, W4, b4,
           LW1, Lb1, LW2, Lb2):
    n, _ = x.shape
    e = edge_index.shape[1]

    bn = 1024
    np_ = ((n + bn - 1) // bn) * bn
    rpt = np_ // NS
    pad = n
    nch = (e + NW * CH - 1) // (NW * CH)
    nch = nch + (nch % 2)

    nch0 = 24
    nch1 = 2 * nch - nch0
    nmax = max(nch0, nch1)

    nq = NW * nch
    src = edge_index[0]
    dst = edge_index[1]
    src3 = (jnp.full(((nq + nmax) * CH,), pad, jnp.int32)
            .at[:e].set(src).reshape(nq + nmax, CH))
    dst3 = (jnp.full(((nq + nmax) * CH,), pad, jnp.int32)
            .at[:e].set(dst).reshape(nq + nmax, CH))

    xp = jnp.pad(x, ((0, np_ - n), (0, 0)))
    batch2 = jnp.pad(batch, (0, np_ - n), constant_values=G).reshape(1, np_)
    b1r = b1.reshape(1, -1)
    b2r = b2.reshape(1, -1)
    b3r = b3.reshape(1, -1)
    b4r = b4.reshape(1, -1)
    lb1r = Lb1.reshape(1, -1)
    lb2r = Lb2.reshape(1, -1)

    z16 = jnp.zeros((rpt, 16), jnp.float32)

    wpad = 128
    prop = _make_prop(np_, nch0, nch1, rpt, wpad)
    z128 = jnp.zeros((rpt, wpad), jnp.float32)

    degp = _make_deg(np_, nch, rpt)(dst3, z16)
    s1, dinv = _tc0(xp, W1, degp[0], degp[1], bn, wpad)

    e1 = prop(src3, dst3, s1, z128)
    s2 = _tc_layer1(e1[0], e1[1], s1, dinv, b1r, bn)

    e2 = prop(src3, dst3, s2, z128)
    s3 = _tc_layer(e2[0], e2[1], s2, dinv, W2, b2r, bn, True, wpad)

    e3 = prop(src3, dst3, s3, z128)
    s4 = _tc_layer(e3[0], e3[1], s3, dinv, W3, b3r, bn, True, wpad)

    e4 = prop(src3, dst3, s4, z128)
    h5 = _tc_layer(e4[0], e4[1], s4, dinv, W4, b4r, bn, False, 256)

    return _tc_head(h5, batch2, LW1, lb1r, LW2, lb2r)


# speedup vs baseline: 2.6306x; 2.6306x over previous
import functools

import jax
import jax.numpy as jnp
from jax import lax
from jax.experimental import pallas as pl
from jax.experimental.pallas import tpu as pltpu
from jax.experimental.pallas import tpu_sc as plsc

NC = 2
NS = 16
NW = NC * NS
CH = 128

G = 64


def _mesh():
    return plsc.VectorSubcoreMesh(
        core_axis_name="c", subcore_axis_name="s", num_cores=NC, num_subcores=NS
    )


def _make_deg(np_, nch, rpt):
    @functools.partial(
        pl.kernel,
        out_type=jax.ShapeDtypeStruct((NC, np_, 16), jnp.float32),
        mesh=_mesh(),
        scratch_types=[
            pltpu.VMEM((nch, CH), jnp.int32),
            pltpu.VMEM((CH, 16), jnp.float32),
            pltpu.VMEM_SHARED((np_, 16), jnp.float32),
        ],
    )
    def deg_kernel(dst_hbm, z_hbm, out_hbm, dst_v, ones_v, acc):
        cid = lax.axis_index("c")
        sid = lax.axis_index("s")
        wid = sid * NC + cid
        pltpu.sync_copy(z_hbm, acc.at[pl.ds(sid * rpt, rpt)])
        pltpu.sync_copy(dst_hbm.at[wid], dst_v)

        def init_ones(i, _):
            ones_v[i, :] = jnp.ones((16,), jnp.float32)
            return 0

        lax.fori_loop(0, CH, init_ones, 0)
        plsc.subcore_barrier()

        def body(j, _):
            pltpu.sync_copy(ones_v, acc.at[dst_v.at[j]], add=True)
            return 0

        lax.fori_loop(0, nch, body, 0)
        plsc.subcore_barrier()
        pltpu.sync_copy(acc.at[pl.ds(sid * rpt, rpt)],
                        out_hbm.at[cid, pl.ds(sid * rpt, rpt)])

    return deg_kernel


def _make_prop(np_, nch, rpt, d):
    @functools.partial(
        pl.kernel,
        out_type=jax.ShapeDtypeStruct((NC, np_, d), jnp.float32),
        mesh=_mesh(),
        scratch_types=[
            pltpu.VMEM((nch, CH), jnp.int32),
            pltpu.VMEM((nch, CH), jnp.int32),
            [pltpu.VMEM((CH, d), jnp.float32) for _ in range(2)],
            pltpu.VMEM_SHARED((np_, d), jnp.float32),
            pltpu.SemaphoreType.DMA,
        ],
    )
    def prop_kernel(src_hbm, dst_hbm, s_hbm, z_hbm, out_hbm,
                    src_v, dst_v, rows, acc, sem):
        cid = lax.axis_index("c")
        sid = lax.axis_index("s")
        wid = sid * NC + cid
        pltpu.sync_copy(z_hbm, acc.at[pl.ds(sid * rpt, rpt)])
        pltpu.sync_copy(src_hbm.at[wid], src_v)
        pltpu.sync_copy(dst_hbm.at[wid], dst_v)
        plsc.subcore_barrier()

        def body(k, _):
            jj = 2 * k
            descs = [
                pltpu.async_copy(s_hbm.at[src_v.at[jj + b]], rows[b], sem)
                for b in range(2)
            ]
            for b in range(2):
                descs[b].wait()
            for b in range(2):
                pltpu.sync_copy(rows[b], acc.at[dst_v.at[jj + b]], add=True)
            return 0

        lax.fori_loop(0, nch // 2, body, 0)
        plsc.subcore_barrier()
        pltpu.sync_copy(acc.at[pl.ds(sid * rpt, rpt)],
                        out_hbm.at[cid, pl.ds(sid * rpt, rpt)])

    return prop_kernel


def _elu(z):
    return jnp.where(z > 0, z, jnp.exp(z) - 1.0)


def _full(shape):
    return pl.BlockSpec(shape, lambda *i: (0,) * len(shape))


def _rows(shape):
    return pl.BlockSpec(shape, lambda i: (i,) + (0,) * (len(shape) - 1))


def _tc0(x, w1, da, db, bn, wpad):
    np_, din = x.shape
    dout = w1.shape[1]

    def body(x_ref, w_ref, da_ref, db_ref, s_ref, dv_ref):
        deg = 1.0 + da_ref[:, :1] + db_ref[:, :1]
        dinv = lax.rsqrt(deg)
        s = dinv * jnp.dot(x_ref[...], w_ref[...],
                           preferred_element_type=jnp.float32)
        s_ref[...] = jnp.concatenate(
            [s, jnp.zeros((bn, wpad - dout), jnp.float32)], axis=1)
        dv_ref[...] = dinv

    return pl.pallas_call(
        body,
        grid=(np_ // bn,),
        in_specs=[_rows((bn, din)), _full(w1.shape), _rows((bn, 16)),
                  _rows((bn, 16))],
        out_specs=[_rows((bn, wpad)), _rows((bn, 1))],
        out_shape=[jax.ShapeDtypeStruct((np_, wpad), jnp.float32),
                   jax.ShapeDtypeStruct((np_, 1), jnp.float32)],
    )(x, w1, da, db)


def _tc_layer1(ea, eb, s1, dinv, b1, bn):
    np_, wpad = s1.shape
    d = b1.shape[1]

    def body(ea_ref, eb_ref, s_ref, dv_ref, b_ref, o_ref):
        dinv_ = dv_ref[...]
        agg = (ea_ref[:, :d] + eb_ref[:, :d] + s_ref[:, :d])
        z = dinv_ * agg + b_ref[...]
        s2 = dinv_ * _elu(z)
        o_ref[...] = jnp.concatenate(
            [s2, jnp.zeros((bn, wpad - d), jnp.float32)], axis=1)

    return pl.pallas_call(
        body,
        grid=(np_ // bn,),
        in_specs=[_rows((bn, wpad)), _rows((bn, wpad)), _rows((bn, wpad)),
                  _rows((bn, 1)), _full(b1.shape)],
        out_specs=_rows((bn, wpad)),
        out_shape=jax.ShapeDtypeStruct((np_, wpad), jnp.float32),
    )(ea, eb, s1, dinv, b1)


def _tc_layer(ea, eb, s, dinv, w, b, bn, scale_out, wpad_out):
    np_, wpad_in = s.shape
    din, dout = w.shape

    def body(ea_ref, eb_ref, s_ref, dv_ref, w_ref, b_ref, o_ref):
        dinv_ = dv_ref[...]
        agg = dinv_ * (ea_ref[:, :din] + eb_ref[:, :din] + s_ref[:, :din])
        z = jnp.dot(agg, w_ref[...], preferred_element_type=jnp.float32) + b_ref[...]
        h = _elu(z)
        h = dinv_ * h if scale_out else h
        if wpad_out > dout:
            h = jnp.concatenate(
                [h, jnp.zeros((bn, wpad_out - dout), jnp.float32)], axis=1)
        o_ref[...] = h

    return pl.pallas_call(
        body,
        grid=(np_ // bn,),
        in_specs=[_rows((bn, wpad_in)), _rows((bn, wpad_in)),
                  _rows((bn, wpad_in)), _rows((bn, 1)), _full(w.shape),
                  _full(b.shape)],
        out_specs=_rows((bn, wpad_out)),
        out_shape=jax.ShapeDtypeStruct((np_, wpad_out), jnp.float32),
    )(ea, eb, s, dinv, w, b)


def _tc_head(h5, batch2, lw1, lb1, lw2, lb2):
    np_, d = h5.shape
    ncls = lw2.shape[1]

    def body(h_ref, b_ref, w1_ref, b1_ref, w2_ref, b2_ref, o_ref):
        gids = lax.broadcasted_iota(jnp.int32, (G, np_), 0)
        onehot = (gids == b_ref[...]).astype(jnp.float32)
        sums = jnp.dot(onehot, h_ref[...], preferred_element_type=jnp.float32)
        cnt = jnp.sum(onehot, axis=1, keepdims=True)
        pooled = sums / jnp.maximum(cnt, 1.0)
        y = _elu(jnp.dot(pooled, w1_ref[...],
                         preferred_element_type=jnp.float32) + b1_ref[...])
        o_ref[...] = jnp.dot(y, w2_ref[...],
                             preferred_element_type=jnp.float32) + b2_ref[...]

    return pl.pallas_call(
        body,
        in_specs=[_full(h5.shape), _full(batch2.shape), _full(lw1.shape),
                  _full(lb1.shape), _full(lw2.shape), _full(lb2.shape)],
        out_specs=_full((G, ncls)),
        out_shape=jax.ShapeDtypeStruct((G, ncls), jnp.float32),
    )(h5, batch2, lw1, lb1, lw2, lb2)


def kernel(x, edge_index, batch, W1, b1, W2, b2, W3, b3, W4, b4,
           LW1, Lb1, LW2, Lb2):
    n, _ = x.shape
    e = edge_index.shape[1]

    bn = 1024
    np_ = ((n + bn - 1) // bn) * bn
    rpt = np_ // NS
    pad = n
    nch = (e + NW * CH - 1) // (NW * CH)
    nch = nch + (nch % 2)
    epad = NW * nch * CH

    src = edge_index[0]
    dst = edge_index[1]
    spread = pad + jnp.arange(epad, dtype=jnp.int32) % (np_ - n - 8)
    src3 = spread.at[:e].set(src).reshape(NW, nch, CH)
    dst3 = spread.at[:e].set(dst).reshape(NW, nch, CH)

    xp = jnp.pad(x, ((0, np_ - n), (0, 0)))
    batch2 = jnp.pad(batch, (0, np_ - n), constant_values=G).reshape(1, np_)
    b1r = b1.reshape(1, -1)
    b2r = b2.reshape(1, -1)
    b3r = b3.reshape(1, -1)
    b4r = b4.reshape(1, -1)
    lb1r = Lb1.reshape(1, -1)
    lb2r = Lb2.reshape(1, -1)

    z16 = jnp.zeros((rpt, 16), jnp.float32)

    wpad = 128
    prop = _make_prop(np_, nch, rpt, wpad)
    z128 = jnp.zeros((rpt, wpad), jnp.float32)

    degp = _make_deg(np_, nch, rpt)(dst3, z16)
    s1, dinv = _tc0(xp, W1, degp[0], degp[1], bn, wpad)

    e1 = prop(src3, dst3, s1, z128)
    s2 = _tc_layer1(e1[0], e1[1], s1, dinv, b1r, bn)

    e2 = prop(src3, dst3, s2, z128)
    s3 = _tc_layer(e2[0], e2[1], s2, dinv, W2, b2r, bn, True, wpad)

    e3 = prop(src3, dst3, s3, z128)
    s4 = _tc_layer(e3[0], e3[1], s3, dinv, W3, b3r, bn, True, wpad)

    e4 = prop(src3, dst3, s4, z128)
    h5 = _tc_layer(e4[0], e4[1], s4, dinv, W4, b4r, bn, False, 256)

    return _tc_head(h5, batch2, LW1, lb1r, LW2, lb2r)
